# baseline (device time: 6306 ns/iter reference)
import jax
import jax.numpy as jnp
from jax import lax
from jax.experimental import pallas as pl
from jax.experimental.pallas import tpu as pltpu

N_BLK = 2


def kernel(x):
    m, n = x.shape
    nb = n // N_BLK

    def body(x_hbm, out_ref, vbuf, comm_ref, copy_sems, send_sems, recv_sems):
        my_x = lax.axis_index("x")
        my_y = lax.axis_index("y")
        peer = (1 - my_x, my_y)

        barrier_sem = pltpu.get_barrier_semaphore()
        pl.semaphore_signal(
            barrier_sem, inc=1,
            device_id=peer, device_id_type=pl.DeviceIdType.MESH,
        )

        copies = []
        for b in range(N_BLK):
            cp = pltpu.make_async_copy(
                x_hbm.at[:, pl.ds(b * nb, nb)],
                vbuf.at[b],
                copy_sems.at[b],
            )
            cp.start()
            copies.append(cp)

        pl.semaphore_wait(barrier_sem, 1)

        rdmas = []
        for b in range(N_BLK):
            copies[b].wait()
            comm_ref[0, :, pl.ds(b * nb, nb)] = jnp.sum(
                vbuf[b], axis=0, keepdims=True
            )
            rdma = pltpu.make_async_remote_copy(
                src_ref=comm_ref.at[0, :, pl.ds(b * nb, nb)],
                dst_ref=comm_ref.at[1, :, pl.ds(b * nb, nb)],
                send_sem=send_sems.at[b],
                recv_sem=recv_sems.at[b],
                device_id=peer,
                device_id_type=pl.DeviceIdType.MESH,
            )
            rdma.start()
            rdmas.append(rdma)

        for b in range(N_BLK):
            rdmas[b].wait()
            out_ref[:, pl.ds(b * nb, nb)] = (
                comm_ref[0, :, pl.ds(b * nb, nb)]
                + comm_ref[1, :, pl.ds(b * nb, nb)]
            )

    out_shape = jax.ShapeDtypeStruct((1, n), jnp.float32)
    return pl.pallas_call(
        body,
        out_shape=out_shape,
        in_specs=[pl.BlockSpec(memory_space=pl.ANY)],
        out_specs=pl.BlockSpec(memory_space=pltpu.VMEM),
        scratch_shapes=[
            pltpu.VMEM((N_BLK, m, nb), jnp.float32),
            pltpu.VMEM((2, 1, n), jnp.float32),
            pltpu.SemaphoreType.DMA((N_BLK,)),
            pltpu.SemaphoreType.DMA((N_BLK,)),
            pltpu.SemaphoreType.DMA((N_BLK,)),
        ],
        compiler_params=pltpu.CompilerParams(collective_id=0),
    )(x)
